# Initial kernel scaffold; baseline (speedup 1.0000x reference)
import functools

import numpy as np
import jax
import jax.numpy as jnp
from jax import lax
from jax.experimental import pallas as pl
from jax.experimental.pallas import tpu as pltpu
from jax.experimental.pallas import tpu_sc as plsc

B_SCALE = 1.3195079565048218
N_LEVELS = 16
BASE_RES = 16
MAX_PARAMS = 2 ** 19
N_POINTS = 524288
P1 = np.int32(np.uint32(2654435761).astype(np.int32))
P2 = np.int32(np.uint32(805459861).astype(np.int32))
HASH_MASK = MAX_PARAMS - 1

NW = 32
PTS_PER_W = N_POINTS // NW
C = 1024
NCHUNK = PTS_PER_W // C
GRP = C // 16


def _level_meta():
    lin, hsh = [], []
    off = 0
    for i in range(N_LEVELS):
        scale = BASE_RES * np.exp(i * np.log(B_SCALE)) - 1.0
        res = int(np.ceil(scale)) + 1
        p = res ** 3
        p = int(p) if p % 8 == 0 else int((p + 7) // 8) * 8
        p = min(MAX_PARAMS, p)
        if res ** 3 <= p:
            lin.append((np.float32(scale), res, p, off))
        else:
            hsh.append((np.float32(scale), off))
        off += p
    return lin, hsh, off


_LIN, _HSH, TOTAL_ROWS = _level_meta()
N_LIN = len(_LIN)
N_HSH = len(_HSH)

_PF = np.zeros((N_LEVELS, 16), np.float32)
_PI = np.zeros((32, 16), np.int32)
_PIH = np.zeros((16, 16), np.int32)
for _l, (_s, _r, _m, _o) in enumerate(_LIN):
    _PF[_l, :] = _s
    _PI[_l, :] = _o
    _PI[8 + _l, :] = _r
    _PI[16 + _l, :] = _m
for _l, (_s, _o) in enumerate(_HSH):
    _PF[N_LIN + _l, :] = _s
    _PIH[_l, :] = _o


def _body(xyzt_hbm, table2_hbm, pi_hbm, pih_hbm, pf_hbm, out_hbm,
          pi_v, pih_v, pf_v, xv, yv, zv, fxv, fyv, fzv,
          idx_v, rows_v, outb_v, sem):
    cc = lax.axis_index("c")
    ss = lax.axis_index("s")
    wid = ss * 2 + cc
    base0 = wid * PTS_PER_W
    ii = lax.iota(jnp.int32, 16)
    zero16 = jnp.zeros((16,), jnp.int32)
    one16 = jnp.ones((16,), jnp.int32)

    pltpu.sync_copy(pi_hbm, pi_v)
    pltpu.sync_copy(pih_hbm, pih_v)
    pltpu.sync_copy(pf_hbm, pf_v)

    def load_pos(g, scale):
        x = xv[pl.ds(g * 16, 16)] * scale + 0.5
        y = yv[pl.ds(g * 16, 16)] * scale + 0.5
        z = zv[pl.ds(g * 16, 16)] * scale + 0.5
        gx = x.astype(jnp.int32)
        gy = y.astype(jnp.int32)
        gz = z.astype(jnp.int32)
        fxv[pl.ds(g * 16, 16)] = x - gx.astype(jnp.float32)
        fyv[pl.ds(g * 16, 16)] = y - gy.astype(jnp.float32)
        fzv[pl.ds(g * 16, 16)] = z - gz.astype(jnp.float32)
        return gx, gy, gz

    def combine_group(g, col0):
        fx = fxv[pl.ds(g * 16, 16)]
        fy = fyv[pl.ds(g * 16, 16)]
        fz = fzv[pl.ds(g * 16, 16)]
        wx0 = 1.0 - fx
        wy0 = 1.0 - fy
        wz0 = 1.0 - fz
        w00 = wx0 * wy0
        w10 = fx * wy0
        w01 = wx0 * fy
        w11 = fx * fy
        ws = (w00 * wz0, w10 * wz0, w01 * wz0, w11 * wz0,
              w00 * fz, w10 * fz, w01 * fz, w11 * fz)
        f0 = jnp.zeros((16,), jnp.float32)
        f1 = jnp.zeros((16,), jnp.float32)
        rbase = g * 128
        for corner in range(8):
            rid = rbase + corner * 16 + ii
            r0 = plsc.load_gather(rows_v, [rid, zero16])
            r1 = plsc.load_gather(rows_v, [rid, one16])
            f0 = f0 + ws[corner] * r0
            f1 = f1 + ws[corner] * r1
        prow = g * 16 + ii
        plsc.store_scatter(outb_v, [prow, col0], f0)
        plsc.store_scatter(outb_v, [prow, col0 + 1], f1)

    def chunk_body(ch, carry):
        base = base0 + ch * C
        pltpu.sync_copy(xyzt_hbm.at[0, pl.ds(base, C)], xv)
        pltpu.sync_copy(xyzt_hbm.at[1, pl.ds(base, C)], yv)
        pltpu.sync_copy(xyzt_hbm.at[2, pl.ds(base, C)], zv)

        def lin_level(l, carry2):
            scale = pf_v[l, :]
            off = pi_v[l, :]
            res = pi_v[8 + l, :]
            m = pi_v[16 + l, :]

            def grp_idx(g, c3):
                gx, gy, gz = load_pos(g, scale)
                x0 = gx
                x1 = gx + 1
                y0 = gy * res
                y1 = y0 + res
                rr = res * res
                z0 = gz * rr
                z1 = z0 + rr
                sb = g * 128
                for corner in range(8):
                    h = ((x1 if corner & 1 else x0)
                         + (y1 if corner & 2 else y0)
                         + (z1 if corner & 4 else z0))
                    h = jnp.where(h >= m, h - m, h)
                    idx_v[pl.ds(sb + corner * 16, 16)] = h + off
                return c3
            lax.fori_loop(0, GRP, grp_idx, 0)
            pltpu.async_copy(table2_hbm.at[idx_v], rows_v, sem).wait()

            col0 = 2 * l + zero16

            def grp_comb(g, c3):
                combine_group(g, col0)
                return c3
            lax.fori_loop(0, GRP, grp_comb, 0)
            return carry2
        lax.fori_loop(0, N_LIN, lin_level, 0)

        def hsh_level(l, carry2):
            scale = pf_v[N_LIN + l, :]
            off = pih_v[l, :]

            def grp_idx(g, c3):
                gx, gy, gz = load_pos(g, scale)
                x0 = gx
                x1 = gx + 1
                y0 = gy * P1
                y1 = y0 + P1
                z0 = gz * P2
                z1 = z0 + P2
                sb = g * 128
                for corner in range(8):
                    h = ((x1 if corner & 1 else x0)
                         ^ (y1 if corner & 2 else y0)
                         ^ (z1 if corner & 4 else z0))
                    h = h & HASH_MASK
                    idx_v[pl.ds(sb + corner * 16, 16)] = h + off
                return c3
            lax.fori_loop(0, GRP, grp_idx, 0)
            pltpu.async_copy(table2_hbm.at[idx_v], rows_v, sem).wait()

            col0 = 2 * (N_LIN + l) + zero16

            def grp_comb(g, c3):
                combine_group(g, col0)
                return c3
            lax.fori_loop(0, GRP, grp_comb, 0)
            return carry2
        lax.fori_loop(0, N_HSH, hsh_level, 0)

        pltpu.sync_copy(outb_v, out_hbm.at[pl.ds(base, C)])
        return carry
    lax.fori_loop(0, NCHUNK, chunk_body, 0)


def _encode(xyzt, table2, pi, pih, pf):
    k = functools.partial(
        pl.kernel,
        mesh=plsc.VectorSubcoreMesh(core_axis_name="c", subcore_axis_name="s"),
        out_type=jax.ShapeDtypeStruct((N_POINTS, 32), jnp.float32),
        scratch_types=[
            pltpu.VMEM((32, 16), jnp.int32),
            pltpu.VMEM((16, 16), jnp.int32),
            pltpu.VMEM((N_LEVELS, 16), jnp.float32),
            pltpu.VMEM((C,), jnp.float32),
            pltpu.VMEM((C,), jnp.float32),
            pltpu.VMEM((C,), jnp.float32),
            pltpu.VMEM((C,), jnp.float32),
            pltpu.VMEM((C,), jnp.float32),
            pltpu.VMEM((C,), jnp.float32),
            pltpu.VMEM((C * 8,), jnp.int32),
            pltpu.VMEM((C * 8, 2), jnp.float32),
            pltpu.VMEM((C, 32), jnp.float32),
            pltpu.SemaphoreType.DMA,
        ],
    )(_body)
    return k(xyzt, table2, pi, pih, pf)


def kernel(xyzs, table):
    xyzt = xyzs.T
    table2 = table.reshape(TOTAL_ROWS, 2)
    pi = jnp.asarray(_PI)
    pih = jnp.asarray(_PIH)
    pf = jnp.asarray(_PF)
    return _encode(xyzt, table2, pi, pih, pf)


# speedup vs baseline: 1.4867x; 1.4867x over previous
import functools

import numpy as np
import jax
import jax.numpy as jnp
from jax import lax
from jax.experimental import pallas as pl
from jax.experimental.pallas import tpu as pltpu
from jax.experimental.pallas import tpu_sc as plsc

B_SCALE = 1.3195079565048218
N_LEVELS = 16
BASE_RES = 16
MAX_PARAMS = 2 ** 19
N_POINTS = 524288
P1 = np.int32(np.uint32(2654435761).astype(np.int32))
P2 = np.int32(np.uint32(805459861).astype(np.int32))
HASH_MASK = MAX_PARAMS - 1

NW = 32
PTS_PER_W = N_POINTS // NW
C = 1024
NCHUNK = PTS_PER_W // C
GRP = C // 16


def _level_meta():
    lin, hsh = [], []
    off = 0
    for i in range(N_LEVELS):
        scale = BASE_RES * np.exp(i * np.log(B_SCALE)) - 1.0
        res = int(np.ceil(scale)) + 1
        p = res ** 3
        p = int(p) if p % 8 == 0 else int((p + 7) // 8) * 8
        p = min(MAX_PARAMS, p)
        if res ** 3 <= p:
            lin.append((np.float32(scale), res, p, off))
        else:
            hsh.append((np.float32(scale), off))
        off += p
    return lin, hsh, off


_LIN, _HSH, TOTAL_ROWS = _level_meta()
N_LIN = len(_LIN)
N_HSH = len(_HSH)

_PF = np.zeros((N_LEVELS * 16,), np.float32)
_PI = np.zeros((24 * 16,), np.int32)
_PIH = np.zeros((16 * 16,), np.int32)
for _l, (_s, _r, _m, _o) in enumerate(_LIN):
    _PF[_l * 16:(_l + 1) * 16] = _s
    _PI[_l * 16:(_l + 1) * 16] = 2 * _o
    _PI[(8 + _l) * 16:(9 + _l) * 16] = _r
    _PI[(16 + _l) * 16:(17 + _l) * 16] = _m
for _l, (_s, _o) in enumerate(_HSH):
    _PF[(N_LIN + _l) * 16:(N_LIN + _l + 1) * 16] = _s
    _PIH[_l * 16:(_l + 1) * 16] = 2 * _o


def _body(x_hbm, y_hbm, z_hbm, table_hbm, pi_hbm, pih_hbm, pf_hbm, out_hbm,
          pi_v, pih_v, pf_v, xv, yv, zv, fxv, fyv, fzv,
          idx0_v, idx1_v, r0_v, r1_v, outb_v, sem):
    cc = lax.axis_index("c")
    ss = lax.axis_index("s")
    wid = ss * 2 + cc
    base0 = wid * PTS_PER_W

    pltpu.sync_copy(pi_hbm, pi_v)
    pltpu.sync_copy(pih_hbm, pih_v)
    pltpu.sync_copy(pf_hbm, pf_v)

    def load_pos(g, scale):
        x = xv[pl.ds(g * 16, 16)] * scale + 0.5
        y = yv[pl.ds(g * 16, 16)] * scale + 0.5
        z = zv[pl.ds(g * 16, 16)] * scale + 0.5
        gx = x.astype(jnp.int32)
        gy = y.astype(jnp.int32)
        gz = z.astype(jnp.int32)
        fxv[pl.ds(g * 16, 16)] = x - gx.astype(jnp.float32)
        fyv[pl.ds(g * 16, 16)] = y - gy.astype(jnp.float32)
        fzv[pl.ds(g * 16, 16)] = z - gz.astype(jnp.float32)
        return gx, gy, gz

    def fire_gathers():
        d0 = pltpu.async_copy(table_hbm.at[idx0_v], r0_v, sem)
        d1 = pltpu.async_copy(table_hbm.at[idx1_v], r1_v, sem)
        d0.wait()
        d1.wait()

    def combine_group(g, col0):
        fx = fxv[pl.ds(g * 16, 16)]
        fy = fyv[pl.ds(g * 16, 16)]
        fz = fzv[pl.ds(g * 16, 16)]
        wx0 = 1.0 - fx
        wy0 = 1.0 - fy
        wz0 = 1.0 - fz
        w00 = wx0 * wy0
        w10 = fx * wy0
        w01 = wx0 * fy
        w11 = fx * fy
        ws = (w00 * wz0, w10 * wz0, w01 * wz0, w11 * wz0,
              w00 * fz, w10 * fz, w01 * fz, w11 * fz)
        f0 = jnp.zeros((16,), jnp.float32)
        f1 = jnp.zeros((16,), jnp.float32)
        rbase = g * 128
        for corner in range(8):
            r0 = r0_v[pl.ds(rbase + corner * 16, 16)]
            r1 = r1_v[pl.ds(rbase + corner * 16, 16)]
            f0 = f0 + ws[corner] * r0
            f1 = f1 + ws[corner] * r1
        outb_v[pl.ds(col0 * C + g * 16, 16)] = f0
        outb_v[pl.ds((col0 + 1) * C + g * 16, 16)] = f1

    def chunk_body(ch, carry):
        base = base0 + ch * C
        pltpu.sync_copy(x_hbm.at[pl.ds(base, C)], xv)
        pltpu.sync_copy(y_hbm.at[pl.ds(base, C)], yv)
        pltpu.sync_copy(z_hbm.at[pl.ds(base, C)], zv)

        def lin_level(l, carry2):
            scale = pf_v[pl.ds(l * 16, 16)]
            off2 = pi_v[pl.ds(l * 16, 16)]
            res = pi_v[pl.ds((8 + l) * 16, 16)]
            m = pi_v[pl.ds((16 + l) * 16, 16)]

            def grp_idx(g, c3):
                gx, gy, gz = load_pos(g, scale)
                x0 = gx
                x1 = gx + 1
                y0 = gy * res
                y1 = y0 + res
                rr = res * res
                z0 = gz * rr
                z1 = z0 + rr
                sb = g * 128
                for corner in range(8):
                    h = ((x1 if corner & 1 else x0)
                         + (y1 if corner & 2 else y0)
                         + (z1 if corner & 4 else z0))
                    h = jnp.where(h >= m, h - m, h)
                    t = off2 + 2 * h
                    idx0_v[pl.ds(sb + corner * 16, 16)] = t
                    idx1_v[pl.ds(sb + corner * 16, 16)] = t + 1
                return c3
            lax.fori_loop(0, GRP, grp_idx, 0)
            fire_gathers()

            col0 = 2 * l

            def grp_comb(g, c3):
                combine_group(g, col0)
                return c3
            lax.fori_loop(0, GRP, grp_comb, 0)
            return carry2
        lax.fori_loop(0, N_LIN, lin_level, 0)

        def hsh_level(l, carry2):
            scale = pf_v[pl.ds((N_LIN + l) * 16, 16)]
            off2 = pih_v[pl.ds(l * 16, 16)]

            def grp_idx(g, c3):
                gx, gy, gz = load_pos(g, scale)
                x0 = gx
                x1 = gx + 1
                y0 = gy * P1
                y1 = y0 + P1
                z0 = gz * P2
                z1 = z0 + P2
                sb = g * 128
                for corner in range(8):
                    h = ((x1 if corner & 1 else x0)
                         ^ (y1 if corner & 2 else y0)
                         ^ (z1 if corner & 4 else z0))
                    h = h & HASH_MASK
                    t = off2 + 2 * h
                    idx0_v[pl.ds(sb + corner * 16, 16)] = t
                    idx1_v[pl.ds(sb + corner * 16, 16)] = t + 1
                return c3
            lax.fori_loop(0, GRP, grp_idx, 0)
            fire_gathers()

            col0 = 2 * (N_LIN + l)

            def grp_comb(g, c3):
                combine_group(g, col0)
                return c3
            lax.fori_loop(0, GRP, grp_comb, 0)
            return carry2
        lax.fori_loop(0, N_HSH, hsh_level, 0)

        descs = [pltpu.async_copy(outb_v.at[pl.ds(col * C, C)],
                                  out_hbm.at[pl.ds(col * N_POINTS + base, C)],
                                  sem)
                 for col in range(32)]
        for d in descs:
            d.wait()
        return carry
    lax.fori_loop(0, NCHUNK, chunk_body, 0)


def _encode(x, y, z, table, pi, pih, pf):
    k = functools.partial(
        pl.kernel,
        mesh=plsc.VectorSubcoreMesh(core_axis_name="c", subcore_axis_name="s"),
        out_type=jax.ShapeDtypeStruct((N_POINTS * 32,), jnp.float32),
        scratch_types=[
            pltpu.VMEM((24 * 16,), jnp.int32),
            pltpu.VMEM((16 * 16,), jnp.int32),
            pltpu.VMEM((N_LEVELS * 16,), jnp.float32),
            pltpu.VMEM((C,), jnp.float32),
            pltpu.VMEM((C,), jnp.float32),
            pltpu.VMEM((C,), jnp.float32),
            pltpu.VMEM((C,), jnp.float32),
            pltpu.VMEM((C,), jnp.float32),
            pltpu.VMEM((C,), jnp.float32),
            pltpu.VMEM((C * 8,), jnp.int32),
            pltpu.VMEM((C * 8,), jnp.int32),
            pltpu.VMEM((C * 8,), jnp.float32),
            pltpu.VMEM((C * 8,), jnp.float32),
            pltpu.VMEM((C * 32,), jnp.float32),
            pltpu.SemaphoreType.DMA,
        ],
    )(_body)
    return k(x, y, z, table, pi, pih, pf)


def kernel(xyzs, table):
    x = xyzs[:, 0]
    y = xyzs[:, 1]
    z = xyzs[:, 2]
    out = _encode(x, y, z, table,
                  jnp.asarray(_PI), jnp.asarray(_PIH), jnp.asarray(_PF))
    return out.reshape(32, N_POINTS).T


# speedup vs baseline: 1.6303x; 1.0966x over previous
import functools

import numpy as np
import jax
import jax.numpy as jnp
from jax import lax
from jax.experimental import pallas as pl
from jax.experimental.pallas import tpu as pltpu
from jax.experimental.pallas import tpu_sc as plsc

B_SCALE = 1.3195079565048218
N_LEVELS = 16
BASE_RES = 16
MAX_PARAMS = 2 ** 19
N_POINTS = 524288
P1 = int(np.uint32(2654435761).astype(np.int32))
P2 = int(np.uint32(805459861).astype(np.int32))
HASH_MASK = MAX_PARAMS - 1

NW = 32
PTS_PER_W = N_POINTS // NW
C = 1024
NCHUNK = PTS_PER_W // C
GRP = C // 16


def _level_meta():
    levels = []
    off = 0
    for i in range(N_LEVELS):
        scale = BASE_RES * np.exp(i * np.log(B_SCALE)) - 1.0
        res = int(np.ceil(scale)) + 1
        p = res ** 3
        p = int(p) if p % 8 == 0 else int((p + 7) // 8) * 8
        p = min(MAX_PARAMS, p)
        levels.append({
            "scale": float(np.float32(scale)),
            "res": res,
            "size": p,
            "off2": 2 * off,
            "dense": res ** 3 <= p,
        })
        off += p
    return levels, off


_LEVELS, TOTAL_ROWS = _level_meta()


def _body(x_hbm, y_hbm, z_hbm, table_hbm, out_hbm,
          xv, yv, zv,
          fxv0, fyv0, fzv0, fxv1, fyv1, fzv1,
          ia0, ib0, ia1, ib1,
          ra0, rb0, ra1, rb1,
          outb_v, sem):
    cc = lax.axis_index("c")
    ss = lax.axis_index("s")
    wid = ss * 2 + cc
    base0 = wid * PTS_PER_W

    fxs = (fxv0, fxv1)
    fys = (fyv0, fyv1)
    fzs = (fzv0, fzv1)
    idx0s = (ia0, ia1)
    idx1s = (ib0, ib1)
    r0s = (ra0, ra1)
    r1s = (rb0, rb1)

    def compute_idx(l, s):
        lv = _LEVELS[l]
        scale = lv["scale"]
        fxv, fyv, fzv = fxs[s], fys[s], fzs[s]
        idx0_v, idx1_v = idx0s[s], idx1s[s]
        off2 = lv["off2"]
        if lv["dense"]:
            res, m = lv["res"], lv["size"]
            rr = res * res
        else:
            res = m = rr = None

        def grp_idx(g, c3):
            x = xv[pl.ds(g * 16, 16)] * scale + 0.5
            y = yv[pl.ds(g * 16, 16)] * scale + 0.5
            z = zv[pl.ds(g * 16, 16)] * scale + 0.5
            gx = x.astype(jnp.int32)
            gy = y.astype(jnp.int32)
            gz = z.astype(jnp.int32)
            fxv[pl.ds(g * 16, 16)] = x - gx.astype(jnp.float32)
            fyv[pl.ds(g * 16, 16)] = y - gy.astype(jnp.float32)
            fzv[pl.ds(g * 16, 16)] = z - gz.astype(jnp.float32)
            if lv["dense"]:
                x0 = gx
                x1 = gx + 1
                y0 = gy * res
                y1 = y0 + res
                z0 = gz * rr
                z1 = z0 + rr
            else:
                x0 = gx
                x1 = gx + 1
                y0 = gy * P1
                y1 = y0 + P1
                z0 = gz * P2
                z1 = z0 + P2
            sb = g * 128
            for corner in range(8):
                cx = x1 if corner & 1 else x0
                cy = y1 if corner & 2 else y0
                cz = z1 if corner & 4 else z0
                if lv["dense"]:
                    h = cx + cy + cz
                    h = jnp.where(h >= m, h - m, h)
                else:
                    h = (cx ^ cy ^ cz) & HASH_MASK
                t = 2 * h + off2
                idx0_v[pl.ds(sb + corner * 16, 16)] = t
                idx1_v[pl.ds(sb + corner * 16, 16)] = t + 1
            return c3
        lax.fori_loop(0, GRP, grp_idx, 0)

    def fire(s):
        d0 = pltpu.async_copy(table_hbm.at[idx0s[s]], r0s[s], sem)
        d1 = pltpu.async_copy(table_hbm.at[idx1s[s]], r1s[s], sem)
        return d0, d1

    def combine(l, s):
        fxv, fyv, fzv = fxs[s], fys[s], fzs[s]
        r0_v, r1_v = r0s[s], r1s[s]
        col0 = 2 * l

        def grp_comb(g, c3):
            fx = fxv[pl.ds(g * 16, 16)]
            fy = fyv[pl.ds(g * 16, 16)]
            fz = fzv[pl.ds(g * 16, 16)]
            wx0 = 1.0 - fx
            wy0 = 1.0 - fy
            wz0 = 1.0 - fz
            w00 = wx0 * wy0
            w10 = fx * wy0
            w01 = wx0 * fy
            w11 = fx * fy
            ws = (w00 * wz0, w10 * wz0, w01 * wz0, w11 * wz0,
                  w00 * fz, w10 * fz, w01 * fz, w11 * fz)
            f0 = jnp.zeros((16,), jnp.float32)
            f1 = jnp.zeros((16,), jnp.float32)
            rbase = g * 128
            for corner in range(8):
                r0 = r0_v[pl.ds(rbase + corner * 16, 16)]
                r1 = r1_v[pl.ds(rbase + corner * 16, 16)]
                f0 = f0 + ws[corner] * r0
                f1 = f1 + ws[corner] * r1
            outb_v[pl.ds(col0 * C + g * 16, 16)] = f0
            outb_v[pl.ds((col0 + 1) * C + g * 16, 16)] = f1
            return c3
        lax.fori_loop(0, GRP, grp_comb, 0)

    def chunk_body(ch, carry):
        base = base0 + ch * C
        pltpu.sync_copy(x_hbm.at[pl.ds(base, C)], xv)
        pltpu.sync_copy(y_hbm.at[pl.ds(base, C)], yv)
        pltpu.sync_copy(z_hbm.at[pl.ds(base, C)], zv)

        compute_idx(0, 0)
        descs = [None, None]
        descs[0] = fire(0)
        for l in range(N_LEVELS):
            cur = l % 2
            nxt = 1 - cur
            if l + 1 < N_LEVELS:
                compute_idx(l + 1, nxt)
            d0, d1 = descs[cur]
            d0.wait()
            d1.wait()
            if l + 1 < N_LEVELS:
                descs[nxt] = fire(nxt)
            combine(l, cur)

        wdescs = [pltpu.async_copy(outb_v.at[pl.ds(col * C, C)],
                                   out_hbm.at[pl.ds(col * N_POINTS + base, C)],
                                   sem)
                  for col in range(32)]
        for d in wdescs:
            d.wait()
        return carry
    lax.fori_loop(0, NCHUNK, chunk_body, 0)


def _encode(x, y, z, table):
    k = functools.partial(
        pl.kernel,
        mesh=plsc.VectorSubcoreMesh(core_axis_name="c", subcore_axis_name="s"),
        out_type=jax.ShapeDtypeStruct((N_POINTS * 32,), jnp.float32),
        scratch_types=[
            pltpu.VMEM((C,), jnp.float32),
            pltpu.VMEM((C,), jnp.float32),
            pltpu.VMEM((C,), jnp.float32),
            pltpu.VMEM((C,), jnp.float32),
            pltpu.VMEM((C,), jnp.float32),
            pltpu.VMEM((C,), jnp.float32),
            pltpu.VMEM((C,), jnp.float32),
            pltpu.VMEM((C,), jnp.float32),
            pltpu.VMEM((C,), jnp.float32),
            pltpu.VMEM((C * 8,), jnp.int32),
            pltpu.VMEM((C * 8,), jnp.int32),
            pltpu.VMEM((C * 8,), jnp.int32),
            pltpu.VMEM((C * 8,), jnp.int32),
            pltpu.VMEM((C * 8,), jnp.float32),
            pltpu.VMEM((C * 8,), jnp.float32),
            pltpu.VMEM((C * 8,), jnp.float32),
            pltpu.VMEM((C * 8,), jnp.float32),
            pltpu.VMEM((C * 32,), jnp.float32),
            pltpu.SemaphoreType.DMA,
        ],
    )(_body)
    return k(x, y, z, table)


def kernel(xyzs, table):
    out = _encode(xyzs[:, 0], xyzs[:, 1], xyzs[:, 2], table)
    return out.reshape(32, N_POINTS).T


# speedup vs baseline: 5.2874x; 3.2432x over previous
import functools

import numpy as np
import jax
import jax.numpy as jnp
from jax import lax
from jax.experimental import pallas as pl
from jax.experimental.pallas import tpu as pltpu
from jax.experimental.pallas import tpu_sc as plsc

B_SCALE = 1.3195079565048218
N_LEVELS = 16
BASE_RES = 16
MAX_PARAMS = 2 ** 19
N_POINTS = 524288
P1 = int(np.uint32(2654435761).astype(np.int32))
P2 = int(np.uint32(805459861).astype(np.int32))
HASH_MASK = MAX_PARAMS - 1

NW = 32
PTS_PER_W = N_POINTS // NW
C = 512
NCHUNK = PTS_PER_W // C
GRP = C // 16
BLK = 8192
SPMEM_WORDS = 2 * MAX_PARAMS
HBLK = 2 * MAX_PARAMS // BLK


def _level_meta():
    levels = []
    off = 0
    for i in range(N_LEVELS):
        scale = BASE_RES * np.exp(i * np.log(B_SCALE)) - 1.0
        res = int(np.ceil(scale)) + 1
        p = res ** 3
        p = int(p) if p % 8 == 0 else int((p + 7) // 8) * 8
        p = min(MAX_PARAMS, p)
        levels.append({
            "scale": float(np.float32(scale)),
            "res": res,
            "size": p,
            "off2": 2 * off,
            "dense": res ** 3 <= p,
        })
        off += p
    return levels, off


_LEVELS, TOTAL_ROWS = _level_meta()
_DENSE = [lv for lv in _LEVELS if lv["dense"]]
_HASHED = [lv for lv in _LEVELS if not lv["dense"]]
ND = len(_DENSE)
NH = len(_HASHED)

_PI = np.zeros((544,), np.int32)
for _l, lv in enumerate(_DENSE):
    _PI[_l * 16:(_l + 1) * 16] = lv["res"]
    _PI[96 + _l * 16:96 + (_l + 1) * 16] = lv["size"]
    _PI[192 + _l * 16:192 + (_l + 1) * 16] = lv["off2"]
    _PI[288 + _l * 16:288 + (_l + 1) * 16] = (2 * lv["size"] + BLK - 1) // BLK
for _l, lv in enumerate(_HASHED):
    _PI[384 + _l * 16:384 + (_l + 1) * 16] = lv["off2"]
_PF = np.zeros((N_LEVELS * 16,), np.float32)
for _l, lv in enumerate(_DENSE + _HASHED):
    _PF[_l * 16:(_l + 1) * 16] = lv["scale"]


def _body(x_hbm, y_hbm, z_hbm, table_hbm, pi_hbm, pf_hbm, out_hbm,
          pi_v, pf_v,
          xv, yv, zv,
          fxv0, fyv0, fzv0, fxv1, fyv1, fzv1,
          ia0, ib0, ia1, ib1,
          ra0, rb0, ra1, rb1,
          ob0, ob1, stage_v, slice_sh, semg):
    cc = lax.axis_index("c")
    ss = lax.axis_index("s")
    wid = ss * 2 + cc
    base0 = wid * PTS_PER_W

    fxs = (fxv0, fxv1)
    fys = (fyv0, fyv1)
    fzs = (fzv0, fzv1)
    idx0s = (ia0, ia1)
    idx1s = (ib0, ib1)
    r0s = (ra0, ra1)
    r1s = (rb0, rb1)

    pltpu.sync_copy(pi_hbm, pi_v)
    pltpu.sync_copy(pf_hbm, pf_v)

    def stage_level(off2s, nblk):
        def stage_k(k, c):
            b = k * 16 + ss

            @pl.when(b < nblk)
            def _stage():
                pltpu.sync_copy(table_hbm.at[pl.ds(off2s + b * BLK, BLK)],
                                stage_v)
                pltpu.sync_copy(stage_v, slice_sh.at[pl.ds(b * BLK, BLK)])
            return c
        lax.fori_loop(0, (nblk + 15) // 16, stage_k, 0)
        plsc.subcore_barrier()

    def compute_idx(dense, scale, res, m, ch, s):
        fxv, fyv, fzv = fxs[s], fys[s], fzs[s]
        idx0_v, idx1_v = idx0s[s], idx1s[s]
        pltpu.sync_copy(x_hbm.at[pl.ds(base0 + ch * C, C)], xv)
        pltpu.sync_copy(y_hbm.at[pl.ds(base0 + ch * C, C)], yv)
        pltpu.sync_copy(z_hbm.at[pl.ds(base0 + ch * C, C)], zv)

        def grp_idx(g, c3):
            x = xv[pl.ds(g * 16, 16)] * scale + 0.5
            y = yv[pl.ds(g * 16, 16)] * scale + 0.5
            z = zv[pl.ds(g * 16, 16)] * scale + 0.5
            gx = x.astype(jnp.int32)
            gy = y.astype(jnp.int32)
            gz = z.astype(jnp.int32)
            fxv[pl.ds(g * 16, 16)] = x - gx.astype(jnp.float32)
            fyv[pl.ds(g * 16, 16)] = y - gy.astype(jnp.float32)
            fzv[pl.ds(g * 16, 16)] = z - gz.astype(jnp.float32)
            x0 = gx
            x1 = gx + 1
            if dense:
                y0 = gy * res
                y1 = y0 + res
                rr = res * res
                z0 = gz * rr
                z1 = z0 + rr
            else:
                y0 = gy * P1
                y1 = y0 + P1
                z0 = gz * P2
                z1 = z0 + P2
            sb = g * 128
            for corner in range(8):
                cx = x1 if corner & 1 else x0
                cy = y1 if corner & 2 else y0
                cz = z1 if corner & 4 else z0
                if dense:
                    h = cx + cy + cz
                    h = jnp.where(h >= m, h - m, h)
                else:
                    h = (cx ^ cy ^ cz) & HASH_MASK
                t = 2 * h
                idx0_v[pl.ds(sb + corner * 16, 16)] = t
                idx1_v[pl.ds(sb + corner * 16, 16)] = t + 1
            return c3
        lax.fori_loop(0, GRP, grp_idx, 0)

    def fire(s):
        pltpu.async_copy(slice_sh.at[idx0s[s]], r0s[s], semg)
        pltpu.async_copy(slice_sh.at[idx1s[s]], r1s[s], semg)

    def wait(s):
        pltpu.make_async_copy(slice_sh.at[idx0s[s]], r0s[s], semg).wait()
        pltpu.make_async_copy(slice_sh.at[idx1s[s]], r1s[s], semg).wait()

    def combine_wb(lidx, ch, s):
        fxv, fyv, fzv = fxs[s], fys[s], fzs[s]
        r0_v, r1_v = r0s[s], r1s[s]

        def grp_comb(g, c3):
            fx = fxv[pl.ds(g * 16, 16)]
            fy = fyv[pl.ds(g * 16, 16)]
            fz = fzv[pl.ds(g * 16, 16)]
            wx0 = 1.0 - fx
            wy0 = 1.0 - fy
            wz0 = 1.0 - fz
            w00 = wx0 * wy0
            w10 = fx * wy0
            w01 = wx0 * fy
            w11 = fx * fy
            ws = (w00 * wz0, w10 * wz0, w01 * wz0, w11 * wz0,
                  w00 * fz, w10 * fz, w01 * fz, w11 * fz)
            f0 = jnp.zeros((16,), jnp.float32)
            f1 = jnp.zeros((16,), jnp.float32)
            rbase = g * 128
            for corner in range(8):
                r0 = r0_v[pl.ds(rbase + corner * 16, 16)]
                r1 = r1_v[pl.ds(rbase + corner * 16, 16)]
                f0 = f0 + ws[corner] * r0
                f1 = f1 + ws[corner] * r1
            ob0[pl.ds(g * 16, 16)] = f0
            ob1[pl.ds(g * 16, 16)] = f1
            return c3
        lax.fori_loop(0, GRP, grp_comb, 0)
        dst = 2 * lidx * N_POINTS + base0 + ch * C
        pltpu.sync_copy(ob0, out_hbm.at[pl.ds(dst, C)])
        pltpu.sync_copy(ob1, out_hbm.at[pl.ds(dst + N_POINTS, C)])

    def run_level(dense, scale, res, m, lidx):
        compute_idx(dense, scale, res, m, 0, 0)
        fire(0)

        def step(ch, cur, nxt):
            compute_idx(dense, scale, res, m, ch + 1, nxt)
            wait(cur)
            fire(nxt)
            combine_wb(lidx, ch, cur)

        def pair(hc, c):
            step(2 * hc, 0, 1)
            step(2 * hc + 1, 1, 0)
            return c
        lax.fori_loop(0, (NCHUNK - 2) // 2, pair, 0)

        step(NCHUNK - 2, 0, 1)
        wait(1)
        combine_wb(lidx, NCHUNK - 1, 1)
        plsc.subcore_barrier()

    def dense_level(l, carry):
        scale = pf_v[pl.ds(l * 16, 16)]
        res = pi_v[pl.ds(l * 16, 16)]
        m = pi_v[pl.ds(96 + l * 16, 16)]
        off2s = pl.multiple_of(pi_v[pl.ds(192 + l * 16, 16)][0], 8)
        nblk = pi_v[pl.ds(288 + l * 16, 16)][0]
        stage_level(off2s, nblk)
        run_level(True, scale, res, m, l)
        return carry
    lax.fori_loop(0, ND, dense_level, 0)

    def hashed_level(l, carry):
        scale = pf_v[pl.ds((ND + l) * 16, 16)]
        off2s = pl.multiple_of(pi_v[pl.ds(384 + l * 16, 16)][0], 8)
        stage_level(off2s, HBLK)
        run_level(False, scale, None, None, ND + l)
        return carry
    lax.fori_loop(0, NH, hashed_level, 0)


def _encode(x, y, z, table, pi, pf):
    k = functools.partial(
        pl.kernel,
        mesh=plsc.VectorSubcoreMesh(core_axis_name="c", subcore_axis_name="s"),
        out_type=jax.ShapeDtypeStruct((N_POINTS * 32,), jnp.float32),
        scratch_types=[
            pltpu.VMEM((544,), jnp.int32),
            pltpu.VMEM((N_LEVELS * 16,), jnp.float32),
            pltpu.VMEM((C,), jnp.float32),
            pltpu.VMEM((C,), jnp.float32),
            pltpu.VMEM((C,), jnp.float32),
            pltpu.VMEM((C,), jnp.float32),
            pltpu.VMEM((C,), jnp.float32),
            pltpu.VMEM((C,), jnp.float32),
            pltpu.VMEM((C,), jnp.float32),
            pltpu.VMEM((C,), jnp.float32),
            pltpu.VMEM((C,), jnp.float32),
            pltpu.VMEM((C * 8,), jnp.int32),
            pltpu.VMEM((C * 8,), jnp.int32),
            pltpu.VMEM((C * 8,), jnp.int32),
            pltpu.VMEM((C * 8,), jnp.int32),
            pltpu.VMEM((C * 8,), jnp.float32),
            pltpu.VMEM((C * 8,), jnp.float32),
            pltpu.VMEM((C * 8,), jnp.float32),
            pltpu.VMEM((C * 8,), jnp.float32),
            pltpu.VMEM((C,), jnp.float32),
            pltpu.VMEM((C,), jnp.float32),
            pltpu.VMEM((BLK,), jnp.float32),
            pltpu.VMEM_SHARED((SPMEM_WORDS,), jnp.float32),
            pltpu.SemaphoreType.DMA,
        ],
    )(_body)
    return k(x, y, z, table, pi, pf)


def kernel(xyzs, table):
    tpad = jnp.concatenate([table, jnp.zeros((BLK,), table.dtype)])
    out = _encode(xyzs[:, 0], xyzs[:, 1], xyzs[:, 2], tpad,
                  jnp.asarray(_PI), jnp.asarray(_PF))
    return out.reshape(32, N_POINTS).T


# speedup vs baseline: 5.9702x; 1.1291x over previous
import functools

import numpy as np
import jax
import jax.numpy as jnp
from jax import lax
from jax.experimental import pallas as pl
from jax.experimental.pallas import tpu as pltpu
from jax.experimental.pallas import tpu_sc as plsc

B_SCALE = 1.3195079565048218
N_LEVELS = 16
BASE_RES = 16
MAX_PARAMS = 2 ** 19
N_POINTS = 524288
P1 = int(np.uint32(2654435761).astype(np.int32))
P2 = int(np.uint32(805459861).astype(np.int32))
HASH_MASK = MAX_PARAMS - 1

NW = 32
PTS_PER_W = N_POINTS // NW
C = 512
NCHUNK = PTS_PER_W // C
GRP = C // 16
BLK = 8192
SPMEM_WORDS = 2 * MAX_PARAMS
HBLK = 2 * MAX_PARAMS // BLK


def _level_meta():
    levels = []
    off = 0
    for i in range(N_LEVELS):
        scale = BASE_RES * np.exp(i * np.log(B_SCALE)) - 1.0
        res = int(np.ceil(scale)) + 1
        p = res ** 3
        p = int(p) if p % 8 == 0 else int((p + 7) // 8) * 8
        p = min(MAX_PARAMS, p)
        levels.append({
            "scale": float(np.float32(scale)),
            "res": res,
            "size": p,
            "off2": 2 * off,
            "dense": res ** 3 <= p,
        })
        off += p
    return levels, off


_LEVELS, TOTAL_ROWS = _level_meta()
_DENSE = [lv for lv in _LEVELS if lv["dense"]]
_HASHED = [lv for lv in _LEVELS if not lv["dense"]]
ND = len(_DENSE)
NH = len(_HASHED)

_PI = np.zeros((544,), np.int32)
for _l, lv in enumerate(_DENSE):
    _PI[_l * 16:(_l + 1) * 16] = lv["res"]
    _PI[96 + _l * 16:96 + (_l + 1) * 16] = lv["size"]
    _PI[192 + _l * 16:192 + (_l + 1) * 16] = lv["off2"]
    _PI[288 + _l * 16:288 + (_l + 1) * 16] = (2 * lv["size"] + BLK - 1) // BLK
for _l, lv in enumerate(_HASHED):
    _PI[384 + _l * 16:384 + (_l + 1) * 16] = lv["off2"]
_PF = np.zeros((N_LEVELS * 16,), np.float32)
for _l, lv in enumerate(_DENSE + _HASHED):
    _PF[_l * 16:(_l + 1) * 16] = lv["scale"]


def _body(x_hbm, y_hbm, z_hbm, table_hbm, pi_hbm, pf_hbm, out_hbm,
          pi_v, pf_v,
          xv0, yv0, zv0, xv1, yv1, zv1,
          fxv0, fyv0, fzv0, fxv1, fyv1, fzv1,
          ia0, ib0, ia1, ib1,
          ra0, rb0, ra1, rb1,
          oba0, obb0, oba1, obb1, stage_v, slice_sh, semg, semx, semo0, semo1):
    cc = lax.axis_index("c")
    ss = lax.axis_index("s")
    wid = ss * 2 + cc
    base0 = wid * PTS_PER_W

    xs = (xv0, xv1)
    ys = (yv0, yv1)
    zs = (zv0, zv1)
    ob0s = (oba0, oba1)
    ob1s = (obb0, obb1)
    semos = (semo0, semo1)
    fxs = (fxv0, fxv1)
    fys = (fyv0, fyv1)
    fzs = (fzv0, fzv1)
    idx0s = (ia0, ia1)
    idx1s = (ib0, ib1)
    r0s = (ra0, ra1)
    r1s = (rb0, rb1)

    pltpu.sync_copy(pi_hbm, pi_v)
    pltpu.sync_copy(pf_hbm, pf_v)

    def stage_level(off2s, nblk):
        def stage_k(k, c):
            b = k * 16 + ss

            @pl.when(b < nblk)
            def _stage():
                pltpu.sync_copy(table_hbm.at[pl.ds(off2s + b * BLK, BLK)],
                                stage_v)
                pltpu.sync_copy(stage_v, slice_sh.at[pl.ds(b * BLK, BLK)])
            return c
        lax.fori_loop(0, (nblk + 15) // 16, stage_k, 0)
        plsc.subcore_barrier()

    def fire_xyz(ch, s):
        pltpu.async_copy(x_hbm.at[pl.ds(base0 + ch * C, C)], xs[s], semx)
        pltpu.async_copy(y_hbm.at[pl.ds(base0 + ch * C, C)], ys[s], semx)
        pltpu.async_copy(z_hbm.at[pl.ds(base0 + ch * C, C)], zs[s], semx)

    def wait_xyz(s):
        pltpu.make_async_copy(x_hbm.at[pl.ds(base0, C)], xs[s], semx).wait()
        pltpu.make_async_copy(y_hbm.at[pl.ds(base0, C)], ys[s], semx).wait()
        pltpu.make_async_copy(z_hbm.at[pl.ds(base0, C)], zs[s], semx).wait()

    def compute_idx(dense, scale, res, m, ch, s):
        fxv, fyv, fzv = fxs[s], fys[s], fzs[s]
        idx0_v, idx1_v = idx0s[s], idx1s[s]
        xv, yv, zv = xs[s], ys[s], zs[s]

        def grp_idx(g, c3):
            x = xv[pl.ds(g * 16, 16)] * scale + 0.5
            y = yv[pl.ds(g * 16, 16)] * scale + 0.5
            z = zv[pl.ds(g * 16, 16)] * scale + 0.5
            gx = x.astype(jnp.int32)
            gy = y.astype(jnp.int32)
            gz = z.astype(jnp.int32)
            fxv[pl.ds(g * 16, 16)] = x - gx.astype(jnp.float32)
            fyv[pl.ds(g * 16, 16)] = y - gy.astype(jnp.float32)
            fzv[pl.ds(g * 16, 16)] = z - gz.astype(jnp.float32)
            x0 = gx
            x1 = gx + 1
            if dense:
                y0 = gy * res
                y1 = y0 + res
                rr = res * res
                z0 = gz * rr
                z1 = z0 + rr
            else:
                y0 = gy * P1
                y1 = y0 + P1
                z0 = gz * P2
                z1 = z0 + P2
            sb = g * 128
            for corner in range(8):
                cx = x1 if corner & 1 else x0
                cy = y1 if corner & 2 else y0
                cz = z1 if corner & 4 else z0
                if dense:
                    h = cx + cy + cz
                    h = jnp.where(h >= m, h - m, h)
                else:
                    h = (cx ^ cy ^ cz) & HASH_MASK
                t = 2 * h
                idx0_v[pl.ds(sb + corner * 16, 16)] = t
                idx1_v[pl.ds(sb + corner * 16, 16)] = t + 1
            return c3
        lax.fori_loop(0, GRP, grp_idx, 0)

    def fire(s):
        pltpu.async_copy(slice_sh.at[idx0s[s]], r0s[s], semg)
        pltpu.async_copy(slice_sh.at[idx1s[s]], r1s[s], semg)

    def wait(s):
        pltpu.make_async_copy(slice_sh.at[idx0s[s]], r0s[s], semg).wait()
        pltpu.make_async_copy(slice_sh.at[idx1s[s]], r1s[s], semg).wait()

    def combine_wb(lidx, ch, s):
        fxv, fyv, fzv = fxs[s], fys[s], fzs[s]
        r0_v, r1_v = r0s[s], r1s[s]
        ob0, ob1 = ob0s[s], ob1s[s]

        def grp_comb(g, c3):
            fx = fxv[pl.ds(g * 16, 16)]
            fy = fyv[pl.ds(g * 16, 16)]
            fz = fzv[pl.ds(g * 16, 16)]
            wx0 = 1.0 - fx
            wy0 = 1.0 - fy
            wz0 = 1.0 - fz
            w00 = wx0 * wy0
            w10 = fx * wy0
            w01 = wx0 * fy
            w11 = fx * fy
            ws = (w00 * wz0, w10 * wz0, w01 * wz0, w11 * wz0,
                  w00 * fz, w10 * fz, w01 * fz, w11 * fz)
            f0 = jnp.zeros((16,), jnp.float32)
            f1 = jnp.zeros((16,), jnp.float32)
            rbase = g * 128
            for corner in range(8):
                r0 = r0_v[pl.ds(rbase + corner * 16, 16)]
                r1 = r1_v[pl.ds(rbase + corner * 16, 16)]
                f0 = f0 + ws[corner] * r0
                f1 = f1 + ws[corner] * r1
            ob0[pl.ds(g * 16, 16)] = f0
            ob1[pl.ds(g * 16, 16)] = f1
            return c3
        lax.fori_loop(0, GRP, grp_comb, 0)
        dst = 2 * lidx * N_POINTS + base0 + ch * C
        semo = semos[s]
        pltpu.async_copy(ob0, out_hbm.at[pl.ds(dst, C)], semo)
        pltpu.async_copy(ob1, out_hbm.at[pl.ds(dst + N_POINTS, C)], semo)

    def wait_ob(s):
        semo = semos[s]
        pltpu.make_async_copy(ob0s[s], out_hbm.at[pl.ds(base0, C)], semo).wait()
        pltpu.make_async_copy(ob1s[s], out_hbm.at[pl.ds(base0, C)], semo).wait()

    def run_level(dense, scale, res, m, lidx):
        pltpu.sync_copy(x_hbm.at[pl.ds(base0, C)], xs[0])
        pltpu.sync_copy(y_hbm.at[pl.ds(base0, C)], ys[0])
        pltpu.sync_copy(z_hbm.at[pl.ds(base0, C)], zs[0])
        compute_idx(dense, scale, res, m, 0, 0)
        fire(0)
        fire_xyz(1, 1)

        def step(ch, cur, nxt, first):
            last_idx = isinstance(ch, int) and ch + 1 == NCHUNK - 1
            wait_xyz(nxt)
            compute_idx(dense, scale, res, m, ch + 1, nxt)
            if not last_idx:
                fire_xyz(ch + 2, cur)
            wait(cur)
            if not first:
                wait_ob(cur)
            fire(nxt)
            combine_wb(lidx, ch, cur)

        step(0, 0, 1, True)
        step(1, 1, 0, True)

        def pair(hc, c):
            ch = 2 * hc
            step(ch, 0, 1, False)
            step(ch + 1, 1, 0, False)
            return c
        lax.fori_loop(1, (NCHUNK - 2) // 2, pair, 0)

        step(NCHUNK - 2, 0, 1, False)
        wait(1)
        wait_ob(1)
        combine_wb(lidx, NCHUNK - 1, 1)
        wait_ob(0)
        wait_ob(1)
        plsc.subcore_barrier()

    def dense_level(l, carry):
        scale = pf_v[pl.ds(l * 16, 16)]
        res = pi_v[pl.ds(l * 16, 16)]
        m = pi_v[pl.ds(96 + l * 16, 16)]
        off2s = pl.multiple_of(pi_v[pl.ds(192 + l * 16, 16)][0], 8)
        nblk = pi_v[pl.ds(288 + l * 16, 16)][0]
        stage_level(off2s, nblk)
        run_level(True, scale, res, m, l)
        return carry
    lax.fori_loop(0, ND, dense_level, 0)

    def hashed_level(l, carry):
        scale = pf_v[pl.ds((ND + l) * 16, 16)]
        off2s = pl.multiple_of(pi_v[pl.ds(384 + l * 16, 16)][0], 8)
        stage_level(off2s, HBLK)
        run_level(False, scale, None, None, ND + l)
        return carry
    lax.fori_loop(0, NH, hashed_level, 0)


def _encode(x, y, z, table, pi, pf):
    k = functools.partial(
        pl.kernel,
        mesh=plsc.VectorSubcoreMesh(core_axis_name="c", subcore_axis_name="s"),
        out_type=jax.ShapeDtypeStruct((N_POINTS * 32,), jnp.float32),
        scratch_types=[
            pltpu.VMEM((544,), jnp.int32),
            pltpu.VMEM((N_LEVELS * 16,), jnp.float32),
            pltpu.VMEM((C,), jnp.float32),
            pltpu.VMEM((C,), jnp.float32),
            pltpu.VMEM((C,), jnp.float32),
            pltpu.VMEM((C,), jnp.float32),
            pltpu.VMEM((C,), jnp.float32),
            pltpu.VMEM((C,), jnp.float32),
            pltpu.VMEM((C,), jnp.float32),
            pltpu.VMEM((C,), jnp.float32),
            pltpu.VMEM((C,), jnp.float32),
            pltpu.VMEM((C,), jnp.float32),
            pltpu.VMEM((C,), jnp.float32),
            pltpu.VMEM((C,), jnp.float32),
            pltpu.VMEM((C * 8,), jnp.int32),
            pltpu.VMEM((C * 8,), jnp.int32),
            pltpu.VMEM((C * 8,), jnp.int32),
            pltpu.VMEM((C * 8,), jnp.int32),
            pltpu.VMEM((C * 8,), jnp.float32),
            pltpu.VMEM((C * 8,), jnp.float32),
            pltpu.VMEM((C * 8,), jnp.float32),
            pltpu.VMEM((C * 8,), jnp.float32),
            pltpu.VMEM((C,), jnp.float32),
            pltpu.VMEM((C,), jnp.float32),
            pltpu.VMEM((C,), jnp.float32),
            pltpu.VMEM((C,), jnp.float32),
            pltpu.VMEM((BLK,), jnp.float32),
            pltpu.VMEM_SHARED((SPMEM_WORDS,), jnp.float32),
            pltpu.SemaphoreType.DMA,
            pltpu.SemaphoreType.DMA,
            pltpu.SemaphoreType.DMA,
            pltpu.SemaphoreType.DMA,
        ],
    )(_body)
    return k(x, y, z, table, pi, pf)


def kernel(xyzs, table):
    tpad = jnp.concatenate([table, jnp.zeros((BLK,), table.dtype)])
    out = _encode(xyzs[:, 0], xyzs[:, 1], xyzs[:, 2], tpad,
                  jnp.asarray(_PI), jnp.asarray(_PF))
    return out.reshape(32, N_POINTS).T


# speedup vs baseline: 6.0612x; 1.0152x over previous
import functools

import numpy as np
import jax
import jax.numpy as jnp
from jax import lax
from jax.experimental import pallas as pl
from jax.experimental.pallas import tpu as pltpu
from jax.experimental.pallas import tpu_sc as plsc

B_SCALE = 1.3195079565048218
N_LEVELS = 16
BASE_RES = 16
MAX_PARAMS = 2 ** 19
N_POINTS = 524288
P1 = int(np.uint32(2654435761).astype(np.int32))
P2 = int(np.uint32(805459861).astype(np.int32))
HASH_MASK = MAX_PARAMS - 1

NW = 32
PTS_PER_W = N_POINTS // NW
C = 512
NCHUNK = PTS_PER_W // C
GRP = C // 16
BLK = 8192
SPMEM_WORDS = 2 * MAX_PARAMS
HBLK = 2 * MAX_PARAMS // BLK


def _level_meta():
    levels = []
    off = 0
    for i in range(N_LEVELS):
        scale = BASE_RES * np.exp(i * np.log(B_SCALE)) - 1.0
        res = int(np.ceil(scale)) + 1
        p = res ** 3
        p = int(p) if p % 8 == 0 else int((p + 7) // 8) * 8
        p = min(MAX_PARAMS, p)
        levels.append({
            "scale": float(np.float32(scale)),
            "res": res,
            "size": p,
            "off2": 2 * off,
            "dense": res ** 3 <= p,
        })
        off += p
    return levels, off


_LEVELS, TOTAL_ROWS = _level_meta()
_DENSE = [lv for lv in _LEVELS if lv["dense"]]
_HASHED = [lv for lv in _LEVELS if not lv["dense"]]
ND = len(_DENSE)
NH = len(_HASHED)

_PI = np.zeros((544,), np.int32)
for _l, lv in enumerate(_DENSE):
    _PI[_l * 16:(_l + 1) * 16] = lv["res"]
    _PI[96 + _l * 16:96 + (_l + 1) * 16] = lv["size"]
    _PI[192 + _l * 16:192 + (_l + 1) * 16] = lv["off2"]
    _PI[288 + _l * 16:288 + (_l + 1) * 16] = (2 * lv["size"] + BLK - 1) // BLK
for _l, lv in enumerate(_HASHED):
    _PI[384 + _l * 16:384 + (_l + 1) * 16] = lv["off2"]
_PF = np.zeros((N_LEVELS * 16,), np.float32)
for _l, lv in enumerate(_DENSE + _HASHED):
    _PF[_l * 16:(_l + 1) * 16] = lv["scale"]


def _body(x_hbm, y_hbm, z_hbm, table_hbm, pi_hbm, pf_hbm, out_hbm,
          pi_v, pf_v,
          xv0, yv0, zv0, xv1, yv1, zv1,
          fxv0, fyv0, fzv0, fxv1, fyv1, fzv1,
          ia0, ib0, ia1, ib1,
          ra0, rb0, ra1, rb1,
          oba0, obb0, oba1, obb1, stage_v, stage_w, slice_sh, semg, semx, semo0, semo1, sems0, sems1):
    cc = lax.axis_index("c")
    ss = lax.axis_index("s")
    wid = ss * 2 + cc
    base0 = wid * PTS_PER_W

    xs = (xv0, xv1)
    ys = (yv0, yv1)
    zs = (zv0, zv1)
    ob0s = (oba0, oba1)
    ob1s = (obb0, obb1)
    semos = (semo0, semo1)
    fxs = (fxv0, fxv1)
    fys = (fyv0, fyv1)
    fzs = (fzv0, fzv1)
    idx0s = (ia0, ia1)
    idx1s = (ib0, ib1)
    r0s = (ra0, ra1)
    r1s = (rb0, rb1)

    pltpu.sync_copy(pi_hbm, pi_v)
    pltpu.sync_copy(pf_hbm, pf_v)

    def stage_level(off2s, nblk):
        def stage_k(k, c):
            b = k * 16 + ss

            @pl.when(b < nblk)
            def _stage():
                pltpu.sync_copy(table_hbm.at[pl.ds(off2s + b * BLK, BLK)],
                                stage_v)
                pltpu.sync_copy(stage_v, slice_sh.at[pl.ds(b * BLK, BLK)])
            return c
        lax.fori_loop(0, (nblk + 15) // 16, stage_k, 0)
        plsc.subcore_barrier()

    def stage_hashed(off2s):
        stages = (stage_v, stage_w)
        semss = (sems0, sems1)
        nround = HBLK // 16

        def hop1(k, s):
            b = k * 16 + ss
            pltpu.async_copy(table_hbm.at[pl.ds(off2s + b * BLK, BLK)],
                             stages[s], semss[s])

        hop1(0, 0)
        for k in range(nround):
            s = k % 2
            pltpu.make_async_copy(table_hbm.at[pl.ds(off2s, BLK)],
                                  stages[s], semss[s]).wait()
            if k + 1 < nround:
                hop1(k + 1, 1 - s)
            b = k * 16 + ss
            pltpu.sync_copy(stages[s], slice_sh.at[pl.ds(b * BLK, BLK)])
        plsc.subcore_barrier()

    def fire_xyz(ch, s):
        pltpu.async_copy(x_hbm.at[pl.ds(base0 + ch * C, C)], xs[s], semx)
        pltpu.async_copy(y_hbm.at[pl.ds(base0 + ch * C, C)], ys[s], semx)
        pltpu.async_copy(z_hbm.at[pl.ds(base0 + ch * C, C)], zs[s], semx)

    def wait_xyz(s):
        pltpu.make_async_copy(x_hbm.at[pl.ds(base0, C)], xs[s], semx).wait()
        pltpu.make_async_copy(y_hbm.at[pl.ds(base0, C)], ys[s], semx).wait()
        pltpu.make_async_copy(z_hbm.at[pl.ds(base0, C)], zs[s], semx).wait()

    def compute_idx(dense, scale, res, m, ch, s):
        fxv, fyv, fzv = fxs[s], fys[s], fzs[s]
        idx0_v, idx1_v = idx0s[s], idx1s[s]
        xv, yv, zv = xs[s], ys[s], zs[s]

        def grp_idx(g, c3):
            x = xv[pl.ds(g * 16, 16)] * scale + 0.5
            y = yv[pl.ds(g * 16, 16)] * scale + 0.5
            z = zv[pl.ds(g * 16, 16)] * scale + 0.5
            gx = x.astype(jnp.int32)
            gy = y.astype(jnp.int32)
            gz = z.astype(jnp.int32)
            fxv[pl.ds(g * 16, 16)] = x - gx.astype(jnp.float32)
            fyv[pl.ds(g * 16, 16)] = y - gy.astype(jnp.float32)
            fzv[pl.ds(g * 16, 16)] = z - gz.astype(jnp.float32)
            x0 = gx
            x1 = gx + 1
            if dense:
                y0 = gy * res
                y1 = y0 + res
                rr = res * res
                z0 = gz * rr
                z1 = z0 + rr
            else:
                y0 = gy * P1
                y1 = y0 + P1
                z0 = gz * P2
                z1 = z0 + P2
            sb = g * 128
            for corner in range(8):
                cx = x1 if corner & 1 else x0
                cy = y1 if corner & 2 else y0
                cz = z1 if corner & 4 else z0
                if dense:
                    h = cx + cy + cz
                    h = jnp.where(h >= m, h - m, h)
                else:
                    h = (cx ^ cy ^ cz) & HASH_MASK
                t = 2 * h
                idx0_v[pl.ds(sb + corner * 16, 16)] = t
                idx1_v[pl.ds(sb + corner * 16, 16)] = t + 1
            return c3
        lax.fori_loop(0, GRP, grp_idx, 0)

    def fire(s):
        pltpu.async_copy(slice_sh.at[idx0s[s]], r0s[s], semg)
        pltpu.async_copy(slice_sh.at[idx1s[s]], r1s[s], semg)

    def wait(s):
        pltpu.make_async_copy(slice_sh.at[idx0s[s]], r0s[s], semg).wait()
        pltpu.make_async_copy(slice_sh.at[idx1s[s]], r1s[s], semg).wait()

    def combine_wb(lidx, ch, s):
        fxv, fyv, fzv = fxs[s], fys[s], fzs[s]
        r0_v, r1_v = r0s[s], r1s[s]
        ob0, ob1 = ob0s[s], ob1s[s]

        def grp_comb(g, c3):
            fx = fxv[pl.ds(g * 16, 16)]
            fy = fyv[pl.ds(g * 16, 16)]
            fz = fzv[pl.ds(g * 16, 16)]
            wx0 = 1.0 - fx
            wy0 = 1.0 - fy
            wz0 = 1.0 - fz
            w00 = wx0 * wy0
            w10 = fx * wy0
            w01 = wx0 * fy
            w11 = fx * fy
            ws = (w00 * wz0, w10 * wz0, w01 * wz0, w11 * wz0,
                  w00 * fz, w10 * fz, w01 * fz, w11 * fz)
            f0 = jnp.zeros((16,), jnp.float32)
            f1 = jnp.zeros((16,), jnp.float32)
            rbase = g * 128
            for corner in range(8):
                r0 = r0_v[pl.ds(rbase + corner * 16, 16)]
                r1 = r1_v[pl.ds(rbase + corner * 16, 16)]
                f0 = f0 + ws[corner] * r0
                f1 = f1 + ws[corner] * r1
            ob0[pl.ds(g * 16, 16)] = f0
            ob1[pl.ds(g * 16, 16)] = f1
            return c3
        lax.fori_loop(0, GRP, grp_comb, 0)
        dst = 2 * lidx * N_POINTS + base0 + ch * C
        semo = semos[s]
        pltpu.async_copy(ob0, out_hbm.at[pl.ds(dst, C)], semo)
        pltpu.async_copy(ob1, out_hbm.at[pl.ds(dst + N_POINTS, C)], semo)

    def wait_ob(s):
        semo = semos[s]
        pltpu.make_async_copy(ob0s[s], out_hbm.at[pl.ds(base0, C)], semo).wait()
        pltpu.make_async_copy(ob1s[s], out_hbm.at[pl.ds(base0, C)], semo).wait()

    def run_level(dense, scale, res, m, lidx):
        pltpu.sync_copy(x_hbm.at[pl.ds(base0, C)], xs[0])
        pltpu.sync_copy(y_hbm.at[pl.ds(base0, C)], ys[0])
        pltpu.sync_copy(z_hbm.at[pl.ds(base0, C)], zs[0])
        compute_idx(dense, scale, res, m, 0, 0)
        fire(0)
        fire_xyz(1, 1)

        def step(ch, cur, nxt, first):
            last_idx = isinstance(ch, int) and ch + 1 == NCHUNK - 1
            wait_xyz(nxt)
            compute_idx(dense, scale, res, m, ch + 1, nxt)
            if not last_idx:
                fire_xyz(ch + 2, cur)
            wait(cur)
            if not first:
                wait_ob(cur)
            fire(nxt)
            combine_wb(lidx, ch, cur)

        step(0, 0, 1, True)
        step(1, 1, 0, True)

        def pair(hc, c):
            ch = 2 * hc
            step(ch, 0, 1, False)
            step(ch + 1, 1, 0, False)
            return c
        lax.fori_loop(1, (NCHUNK - 2) // 2, pair, 0)

        step(NCHUNK - 2, 0, 1, False)
        wait(1)
        wait_ob(1)
        combine_wb(lidx, NCHUNK - 1, 1)
        wait_ob(0)
        wait_ob(1)
        plsc.subcore_barrier()

    def dense_level(l, carry):
        scale = pf_v[pl.ds(l * 16, 16)]
        res = pi_v[pl.ds(l * 16, 16)]
        m = pi_v[pl.ds(96 + l * 16, 16)]
        off2s = pl.multiple_of(pi_v[pl.ds(192 + l * 16, 16)][0], 8)
        nblk = pi_v[pl.ds(288 + l * 16, 16)][0]
        stage_level(off2s, nblk)
        run_level(True, scale, res, m, l)
        return carry
    lax.fori_loop(0, ND, dense_level, 0)

    def hashed_level(l, carry):
        scale = pf_v[pl.ds((ND + l) * 16, 16)]
        off2s = pl.multiple_of(pi_v[pl.ds(384 + l * 16, 16)][0], 8)
        stage_hashed(off2s)
        run_level(False, scale, None, None, ND + l)
        return carry
    lax.fori_loop(0, NH, hashed_level, 0)


def _encode(x, y, z, table, pi, pf):
    k = functools.partial(
        pl.kernel,
        mesh=plsc.VectorSubcoreMesh(core_axis_name="c", subcore_axis_name="s"),
        out_type=jax.ShapeDtypeStruct((N_POINTS * 32,), jnp.float32),
        scratch_types=[
            pltpu.VMEM((544,), jnp.int32),
            pltpu.VMEM((N_LEVELS * 16,), jnp.float32),
            pltpu.VMEM((C,), jnp.float32),
            pltpu.VMEM((C,), jnp.float32),
            pltpu.VMEM((C,), jnp.float32),
            pltpu.VMEM((C,), jnp.float32),
            pltpu.VMEM((C,), jnp.float32),
            pltpu.VMEM((C,), jnp.float32),
            pltpu.VMEM((C,), jnp.float32),
            pltpu.VMEM((C,), jnp.float32),
            pltpu.VMEM((C,), jnp.float32),
            pltpu.VMEM((C,), jnp.float32),
            pltpu.VMEM((C,), jnp.float32),
            pltpu.VMEM((C,), jnp.float32),
            pltpu.VMEM((C * 8,), jnp.int32),
            pltpu.VMEM((C * 8,), jnp.int32),
            pltpu.VMEM((C * 8,), jnp.int32),
            pltpu.VMEM((C * 8,), jnp.int32),
            pltpu.VMEM((C * 8,), jnp.float32),
            pltpu.VMEM((C * 8,), jnp.float32),
            pltpu.VMEM((C * 8,), jnp.float32),
            pltpu.VMEM((C * 8,), jnp.float32),
            pltpu.VMEM((C,), jnp.float32),
            pltpu.VMEM((C,), jnp.float32),
            pltpu.VMEM((C,), jnp.float32),
            pltpu.VMEM((C,), jnp.float32),
            pltpu.VMEM((BLK,), jnp.float32),
            pltpu.VMEM((BLK,), jnp.float32),
            pltpu.VMEM_SHARED((SPMEM_WORDS,), jnp.float32),
            pltpu.SemaphoreType.DMA,
            pltpu.SemaphoreType.DMA,
            pltpu.SemaphoreType.DMA,
            pltpu.SemaphoreType.DMA,
            pltpu.SemaphoreType.DMA,
            pltpu.SemaphoreType.DMA,
        ],
    )(_body)
    return k(x, y, z, table, pi, pf)


def kernel(xyzs, table):
    tpad = jnp.concatenate([table, jnp.zeros((BLK,), table.dtype)])
    out = _encode(xyzs[:, 0], xyzs[:, 1], xyzs[:, 2], tpad,
                  jnp.asarray(_PI), jnp.asarray(_PF))
    return out.reshape(32, N_POINTS).T
